# baseline (speedup 1.0000x reference)
import functools

import jax
import jax.numpy as jnp
from jax.experimental import pallas as pl
from jax.experimental.pallas import tpu as pltpu

_CHUNK = 512
_NBUF = 4


def _router_kernel(x_hbm, w_ref, o_ref, *scratch):
    xbufs = scratch[:_NBUF]
    in_sems = scratch[_NBUF]
    m = x_hbm.shape[0]
    n_chunks = m // _CHUNK

    def in_copy(i):
        slot = i % _NBUF
        return pltpu.make_async_copy(
            x_hbm.at[pl.ds(i * _CHUNK, _CHUNK), :],
            xbufs[slot],
            in_sems.at[slot],
        )

    wb = w_ref[...].astype(jnp.bfloat16)

    for s in range(min(_NBUF, n_chunks)):
        in_copy(s).start()

    for i in range(n_chunks):
        in_copy(i).wait()
        slot = i % _NBUF
        o_ref[pl.ds(i * _CHUNK, _CHUNK), :] = jax.lax.dot_general(
            xbufs[slot][...].astype(jnp.bfloat16),
            wb,
            (((1,), (1,)), ((), ())),
            preferred_element_type=jnp.float32,
        )
        if i + _NBUF < n_chunks:
            in_copy(i + _NBUF).start()


@functools.partial(jax.jit, static_argnames=())
def kernel(x, gate_w):
    b, t, d = x.shape
    e = gate_w.shape[0]
    m = b * t
    x2 = x.reshape(m, d)

    out = pl.pallas_call(
        _router_kernel,
        in_specs=[
            pl.BlockSpec(memory_space=pl.ANY),
            pl.BlockSpec(memory_space=pltpu.VMEM),
        ],
        out_specs=pl.BlockSpec(memory_space=pltpu.VMEM),
        out_shape=jax.ShapeDtypeStruct((m, e), jnp.float32),
        scratch_shapes=(
            [pltpu.VMEM((_CHUNK, d), jnp.float32) for _ in range(_NBUF)]
            + [pltpu.SemaphoreType.DMA((_NBUF,))]
        ),
    )(x2, gate_w)
    return out.reshape(b, t, e)


# speedup vs baseline: 1.0582x; 1.0582x over previous
import functools

import jax
import jax.numpy as jnp
from jax.experimental import pallas as pl
from jax.experimental.pallas import tpu as pltpu

_BLOCK_M = 1024


def _router_kernel(x_ref, w_ref, o_ref):
    o_ref[...] = jax.lax.dot_general(
        x_ref[...].astype(jnp.bfloat16),
        w_ref[...].astype(jnp.bfloat16),
        (((1,), (1,)), ((), ())),
        preferred_element_type=jnp.float32,
    )


@functools.partial(jax.jit, static_argnames=())
def kernel(x, gate_w):
    b, t, d = x.shape
    e = gate_w.shape[0]
    m = b * t
    x2 = x.reshape(m, d)

    out = pl.pallas_call(
        _router_kernel,
        grid=(m // _BLOCK_M,),
        in_specs=[
            pl.BlockSpec((_BLOCK_M, d), lambda i: (i, 0)),
            pl.BlockSpec((e, d), lambda i: (0, 0)),
        ],
        out_specs=pl.BlockSpec((_BLOCK_M, e), lambda i: (i, 0)),
        out_shape=jax.ShapeDtypeStruct((m, e), jnp.float32),
        compiler_params=pltpu.CompilerParams(
            dimension_semantics=("arbitrary",),
        ),
    )(x2, gate_w)
    return out.reshape(b, t, e)
